# baseline (speedup 1.0000x reference)
import functools

import jax
import jax.numpy as jnp
from jax import lax
from jax.experimental import pallas as pl
from jax.experimental.pallas import tpu as pltpu
from jax.experimental.pallas import tpu_sc as plsc

_STUFF_THRESH = 10
_INST_BASE = 11
_L = 16
_NC = 2
_NS = 16
_NW = _NC * _NS
_TR = 8
_TC = 128

_SC_ROWS = 448
_SC_B0 = 12
_SC_B1 = 8
_RB = 64


@functools.cache
def _make_sc_call(num_gt, H, W, row0, rows):
    col_tiles = W // _TC
    total_tiles = (rows // _TR) * col_tiles
    nchunk = total_tiles // _NW
    assert total_tiles % _NW == 0 and nchunk % 2 == 0

    mesh = plsc.VectorSubcoreMesh(
        core_axis_name="c", subcore_axis_name="s",
        num_cores=_NC, num_subcores=_NS,
    )

    b0 = _SC_B0
    lo0 = num_gt - b0
    b1 = _SC_B1
    b2 = lo0 - b1
    nv = _TR * (_TC // _L)

    def body(segs_hbm, masks_hbm, out_hbm, mb0, segs_v, out_v, mt,
             si0, si1, so0, so1, st):
        wid = lax.axis_index("s") * _NC + lax.axis_index("c")
        t0 = wid * nchunk
        in_sems = (si0, si1)
        out_sems = (so0, so1)

        def tile_origin(k):
            t = t0 + k
            rb = t // col_tiles
            ct = t % col_tiles
            return rb * _TR, ct * _TC

        def in_copies(k, b):
            r0, c0 = tile_origin(k)
            cps = [
                pltpu.make_async_copy(
                    masks_hbm.at[lo0 + j, pl.ds(row0 + r0, _TR), pl.ds(c0, _TC)],
                    mb0.at[b, j],
                    in_sems[b],
                )
                for j in range(b0)
            ]
            cps.append(
                pltpu.make_async_copy(
                    segs_hbm.at[0, pl.ds(row0 + r0, _TR), pl.ds(c0, _TC)],
                    segs_v.at[b],
                    in_sems[b],
                )
            )
            return cps

        def tail_copies(k, lo, n):
            r0, c0 = tile_origin(k)
            return [
                pltpu.make_async_copy(
                    masks_hbm.at[lo + j, pl.ds(row0 + r0, _TR), pl.ds(c0, _TC)],
                    mt.at[j],
                    st,
                )
                for j in range(n)
            ]

        def out_copy(k, b):
            r0, c0 = tile_origin(k)
            return pltpu.make_async_copy(
                out_v.at[b],
                out_hbm.at[0, pl.ds(r0, _TR), pl.ds(c0, _TC)],
                out_sems[b],
            )

        def overwritable(a):
            return (a < _INST_BASE) | (a >= _INST_BASE + num_gt)

        def run_tail(k, b, lo, n):
            cps = tail_copies(k, lo, n)
            for cp in cps:
                cp.start()
            for cp in cps:
                cp.wait()

            def inner_t(v, mn):
                r = v // (_TC // _L)
                off = (v % (_TC // _L)) * _L
                best = jnp.full((_L,), -1, jnp.int32)
                for j in range(n):
                    m = mt[j, r, pl.ds(off, _L)]
                    best = jnp.maximum(
                        best, jnp.where(m != 0.0, lo + j + _INST_BASE, -1)
                    )
                a = out_v[b, r, pl.ds(off, _L)]
                a = jnp.where(overwritable(a) & (best >= 0), best, a)
                out_v[b, r, pl.ds(off, _L)] = a
                return jnp.minimum(
                    mn, jnp.where(overwritable(a), -1, 0)
                )

            return lax.fori_loop(0, nv, inner_t, jnp.zeros((_L,), jnp.int32))

        for cp in in_copies(0, 0):
            cp.start()
        for cp in in_copies(1, 1):
            cp.start()

        def pair(p, carry):
            k0 = p * 2
            for b in range(2):
                k = k0 + b
                for cp in in_copies(k, b):
                    cp.wait()

                @pl.when(k >= 2)
                def _():
                    out_copy(k, b).wait()

                def inner0(v, mn):
                    r = v // (_TC // _L)
                    off = (v % (_TC // _L)) * _L
                    acc = jnp.full((_L,), -1, jnp.int32)
                    for j in range(b0):
                        m = mb0[b, j, r, pl.ds(off, _L)]
                        acc = jnp.where(m != 0.0, lo0 + j, acc)
                    seg = segs_v[b, r, pl.ds(off, _L)]
                    stuff = jnp.where(
                        (seg <= _STUFF_THRESH) | (seg >= 255), seg, 255
                    )
                    out_v[b, r, pl.ds(off, _L)] = jnp.where(
                        acc >= 0, acc + _INST_BASE, stuff
                    )
                    return jnp.minimum(mn, acc)

                mn0 = lax.fori_loop(
                    0, nv, inner0, jnp.zeros((_L,), jnp.int32)
                )

                @pl.when(jnp.min(mn0) < 0)
                def _():
                    mn1 = run_tail(k, b, b2, b1)

                    @pl.when(jnp.min(mn1) < 0)
                    def _():
                        run_tail(k, b, 0, b2)

                out_copy(k, b).start()

                @pl.when(k + 2 < nchunk)
                def _():
                    for cp in in_copies(k + 2, b):
                        cp.start()
            return carry

        lax.fori_loop(0, nchunk // 2, pair, 0)

        for b in range(2):
            out_copy(0, b).wait()

    return pl.kernel(
        body,
        out_type=jax.ShapeDtypeStruct((1, rows, W), jnp.int32),
        mesh=mesh,
        scratch_types=[
            pltpu.VMEM((2, b0, _TR, _TC), jnp.float32),
            pltpu.VMEM((2, _TR, _TC), jnp.int32),
            pltpu.VMEM((2, _TR, _TC), jnp.int32),
            pltpu.VMEM((max(b1, b2), _TR, _TC), jnp.float32),
            pltpu.SemaphoreType.DMA,
            pltpu.SemaphoreType.DMA,
            pltpu.SemaphoreType.DMA,
            pltpu.SemaphoreType.DMA,
            pltpu.SemaphoreType.DMA,
        ],
        compiler_params=pltpu.CompilerParams(
            use_tc_tiling_on_sc=True, needs_layout_passes=False
        ),
    )


_TOP_HALF = 16
_TOP_QTR = 4
_TAIL = 12


@functools.cache
def _make_tc_call(num_gt, H, W, rows):
    assert num_gt == _TOP_HALF + _TOP_QTR + _TAIL
    grid = (rows // _RB,)

    def body(segs_ref, masks_a_ref, masks_b_ref, masks_any, out_ref,
             acc_ref, mbuf, sem):
        acc = jnp.full((_RB, W), -1, jnp.int32)
        for j in range(_TOP_HALF):
            i = _TOP_HALF + j
            acc = jnp.maximum(acc, jnp.where(masks_a_ref[j] != 0.0, i, -1))
        for j in range(_TOP_QTR):
            i = _TAIL + j
            acc = jnp.maximum(acc, jnp.where(masks_b_ref[j] != 0.0, i, -1))
        acc_ref[...] = acc

        @pl.when(jnp.min(acc) < 0)
        def _():
            blk = pl.program_id(0)
            cp = pltpu.make_async_copy(
                masks_any.at[pl.ds(0, _TAIL), pl.ds(blk * _RB, _RB), :],
                mbuf,
                sem,
            )
            cp.start()
            cp.wait()
            a = acc_ref[...]
            for j in range(_TAIL):
                a = jnp.maximum(a, jnp.where(mbuf[j] != 0.0, j, -1))
            acc_ref[...] = a

        accf = acc_ref[...]
        seg = segs_ref[0]
        stuff = jnp.where((seg <= _STUFF_THRESH) | (seg >= 255), seg, 255)
        out_ref[0] = jnp.where(accf >= 0, accf + _INST_BASE, stuff)

    return pl.pallas_call(
        body,
        grid=grid,
        in_specs=[
            pl.BlockSpec((1, _RB, W), lambda i: (0, i, 0)),
            pl.BlockSpec((_TOP_HALF, _RB, W), lambda i: (1, i, 0)),
            pl.BlockSpec((_TOP_QTR, _RB, W), lambda i: (3, i, 0)),
            pl.BlockSpec(memory_space=pl.ANY),
        ],
        out_specs=pl.BlockSpec((1, _RB, W), lambda i: (0, i, 0)),
        out_shape=jax.ShapeDtypeStruct((1, rows, W), jnp.int32),
        scratch_shapes=[
            pltpu.VMEM((_RB, W), jnp.int32),
            pltpu.VMEM((_TAIL, _RB, W), jnp.float32),
            pltpu.SemaphoreType.DMA,
        ],
        compiler_params=pltpu.CompilerParams(
            dimension_semantics=("arbitrary",),
        ),
    )


def kernel(gt_segs, gt_masks):
    _, H, W = gt_segs.shape
    num_gt = gt_masks.shape[0]
    sc_rows = _SC_ROWS
    tc_rows = H - sc_rows
    out_tc = _make_tc_call(num_gt, H, W, tc_rows)(
        gt_segs, gt_masks, gt_masks, gt_masks
    )
    if sc_rows == 0:
        return out_tc
    out_sc = _make_sc_call(num_gt, H, W, tc_rows, sc_rows)(gt_segs, gt_masks)
    return jnp.concatenate([out_tc, out_sc], axis=1)


# speedup vs baseline: 1.1209x; 1.1209x over previous
import functools

import jax
import jax.numpy as jnp
from jax import lax
from jax.experimental import pallas as pl
from jax.experimental.pallas import tpu as pltpu
from jax.experimental.pallas import tpu_sc as plsc

_STUFF_THRESH = 10
_INST_BASE = 11
_L = 16
_NC = 2
_NS = 16
_NW = _NC * _NS
_TR = 8
_TC = 128

_SC_ROWS = 320
_SC_B0 = 12
_SC_B1 = 8
_RB = 64


@functools.cache
def _make_sc_call(num_gt, H, W, row0, rows):
    col_tiles = W // _TC
    total_tiles = (rows // _TR) * col_tiles
    nchunk = total_tiles // _NW
    assert total_tiles % _NW == 0 and nchunk % 2 == 0

    mesh = plsc.VectorSubcoreMesh(
        core_axis_name="c", subcore_axis_name="s",
        num_cores=_NC, num_subcores=_NS,
    )

    b0 = _SC_B0
    lo0 = num_gt - b0
    b1 = _SC_B1
    b2 = lo0 - b1
    nv = _TR * (_TC // _L)

    def body(segs_hbm, masks_hbm, out_hbm, mb0, segs_v, out_v, mt,
             si0, si1, so0, so1, st):
        wid = lax.axis_index("s") * _NC + lax.axis_index("c")
        t0 = wid * nchunk
        in_sems = (si0, si1)
        out_sems = (so0, so1)

        def tile_origin(k):
            t = t0 + k
            rb = t // col_tiles
            ct = t % col_tiles
            return rb * _TR, ct * _TC

        def in_copies(k, b):
            r0, c0 = tile_origin(k)
            cps = [
                pltpu.make_async_copy(
                    masks_hbm.at[lo0 + j, pl.ds(row0 + r0, _TR), pl.ds(c0, _TC)],
                    mb0.at[b, j],
                    in_sems[b],
                )
                for j in range(b0)
            ]
            cps.append(
                pltpu.make_async_copy(
                    segs_hbm.at[0, pl.ds(row0 + r0, _TR), pl.ds(c0, _TC)],
                    segs_v.at[b],
                    in_sems[b],
                )
            )
            return cps

        def tail_copies(k, lo, n):
            r0, c0 = tile_origin(k)
            return [
                pltpu.make_async_copy(
                    masks_hbm.at[lo + j, pl.ds(row0 + r0, _TR), pl.ds(c0, _TC)],
                    mt.at[j],
                    st,
                )
                for j in range(n)
            ]

        def out_copy(k, b):
            r0, c0 = tile_origin(k)
            return pltpu.make_async_copy(
                out_v.at[b],
                out_hbm.at[0, pl.ds(r0, _TR), pl.ds(c0, _TC)],
                out_sems[b],
            )

        def overwritable(a):
            return (a < _INST_BASE) | (a >= _INST_BASE + num_gt)

        def run_tail(k, b, lo, n):
            cps = tail_copies(k, lo, n)
            for cp in cps:
                cp.start()
            for cp in cps:
                cp.wait()

            def inner_t(v, mn):
                r = v // (_TC // _L)
                off = (v % (_TC // _L)) * _L
                best = jnp.full((_L,), -1, jnp.int32)
                for j in range(n):
                    m = mt[j, r, pl.ds(off, _L)]
                    best = jnp.maximum(
                        best, jnp.where(m != 0.0, lo + j + _INST_BASE, -1)
                    )
                a = out_v[b, r, pl.ds(off, _L)]
                a = jnp.where(overwritable(a) & (best >= 0), best, a)
                out_v[b, r, pl.ds(off, _L)] = a
                return jnp.minimum(
                    mn, jnp.where(overwritable(a), -1, 0)
                )

            return lax.fori_loop(0, nv, inner_t, jnp.zeros((_L,), jnp.int32))

        for cp in in_copies(0, 0):
            cp.start()
        for cp in in_copies(1, 1):
            cp.start()

        def pair(p, carry):
            k0 = p * 2
            for b in range(2):
                k = k0 + b
                for cp in in_copies(k, b):
                    cp.wait()

                @pl.when(k >= 2)
                def _():
                    out_copy(k, b).wait()

                def inner0(v, mn):
                    r = v // (_TC // _L)
                    off = (v % (_TC // _L)) * _L
                    acc = jnp.full((_L,), -1, jnp.int32)
                    for j in range(b0):
                        m = mb0[b, j, r, pl.ds(off, _L)]
                        acc = jnp.where(m != 0.0, lo0 + j, acc)
                    seg = segs_v[b, r, pl.ds(off, _L)]
                    stuff = jnp.where(
                        (seg <= _STUFF_THRESH) | (seg >= 255), seg, 255
                    )
                    out_v[b, r, pl.ds(off, _L)] = jnp.where(
                        acc >= 0, acc + _INST_BASE, stuff
                    )
                    return jnp.minimum(mn, acc)

                mn0 = lax.fori_loop(
                    0, nv, inner0, jnp.zeros((_L,), jnp.int32)
                )

                @pl.when(jnp.min(mn0) < 0)
                def _():
                    mn1 = run_tail(k, b, b2, b1)

                    @pl.when(jnp.min(mn1) < 0)
                    def _():
                        run_tail(k, b, 0, b2)

                out_copy(k, b).start()

                @pl.when(k + 2 < nchunk)
                def _():
                    for cp in in_copies(k + 2, b):
                        cp.start()
            return carry

        lax.fori_loop(0, nchunk // 2, pair, 0)

        for b in range(2):
            out_copy(0, b).wait()

    return pl.kernel(
        body,
        out_type=jax.ShapeDtypeStruct((1, rows, W), jnp.int32),
        mesh=mesh,
        scratch_types=[
            pltpu.VMEM((2, b0, _TR, _TC), jnp.float32),
            pltpu.VMEM((2, _TR, _TC), jnp.int32),
            pltpu.VMEM((2, _TR, _TC), jnp.int32),
            pltpu.VMEM((max(b1, b2), _TR, _TC), jnp.float32),
            pltpu.SemaphoreType.DMA,
            pltpu.SemaphoreType.DMA,
            pltpu.SemaphoreType.DMA,
            pltpu.SemaphoreType.DMA,
            pltpu.SemaphoreType.DMA,
        ],
        compiler_params=pltpu.CompilerParams(
            use_tc_tiling_on_sc=True, needs_layout_passes=False
        ),
    )


_TOP_HALF = 16
_TOP_QTR = 4
_TAIL = 12


@functools.cache
def _make_tc_call(num_gt, H, W, rows):
    assert num_gt == _TOP_HALF + _TOP_QTR + _TAIL
    grid = (rows // _RB,)

    def body(segs_ref, masks_a_ref, masks_b_ref, masks_any, out_ref,
             acc_ref, mbuf, sem):
        acc = jnp.full((_RB, W), -1, jnp.int32)
        for j in range(_TOP_HALF):
            i = _TOP_HALF + j
            acc = jnp.maximum(acc, jnp.where(masks_a_ref[j] != 0.0, i, -1))
        for j in range(_TOP_QTR):
            i = _TAIL + j
            acc = jnp.maximum(acc, jnp.where(masks_b_ref[j] != 0.0, i, -1))
        acc_ref[...] = acc

        @pl.when(jnp.min(acc) < 0)
        def _():
            blk = pl.program_id(0)
            cp = pltpu.make_async_copy(
                masks_any.at[pl.ds(0, _TAIL), pl.ds(blk * _RB, _RB), :],
                mbuf,
                sem,
            )
            cp.start()
            cp.wait()
            a = acc_ref[...]
            for j in range(_TAIL):
                a = jnp.maximum(a, jnp.where(mbuf[j] != 0.0, j, -1))
            acc_ref[...] = a

        accf = acc_ref[...]
        seg = segs_ref[0]
        stuff = jnp.where((seg <= _STUFF_THRESH) | (seg >= 255), seg, 255)
        out_ref[0] = jnp.where(accf >= 0, accf + _INST_BASE, stuff)

    return pl.pallas_call(
        body,
        grid=grid,
        in_specs=[
            pl.BlockSpec((1, _RB, W), lambda i: (0, i, 0)),
            pl.BlockSpec((_TOP_HALF, _RB, W), lambda i: (1, i, 0)),
            pl.BlockSpec((_TOP_QTR, _RB, W), lambda i: (3, i, 0)),
            pl.BlockSpec(memory_space=pl.ANY),
        ],
        out_specs=pl.BlockSpec((1, _RB, W), lambda i: (0, i, 0)),
        out_shape=jax.ShapeDtypeStruct((1, rows, W), jnp.int32),
        scratch_shapes=[
            pltpu.VMEM((_RB, W), jnp.int32),
            pltpu.VMEM((_TAIL, _RB, W), jnp.float32),
            pltpu.SemaphoreType.DMA,
        ],
        compiler_params=pltpu.CompilerParams(
            dimension_semantics=("arbitrary",),
        ),
    )


def kernel(gt_segs, gt_masks):
    _, H, W = gt_segs.shape
    num_gt = gt_masks.shape[0]
    sc_rows = _SC_ROWS
    tc_rows = H - sc_rows
    out_tc = _make_tc_call(num_gt, H, W, tc_rows)(
        gt_segs, gt_masks, gt_masks, gt_masks
    )
    if sc_rows == 0:
        return out_tc
    out_sc = _make_sc_call(num_gt, H, W, tc_rows, sc_rows)(gt_segs, gt_masks)
    return jnp.concatenate([out_tc, out_sc], axis=1)
